# baseline (speedup 1.0000x reference)
import jax
import jax.numpy as jnp
from jax import lax
from jax.experimental import pallas as pl
from jax.experimental.pallas import tpu as pltpu
from jax.experimental.pallas import tpu_sc as plsc

_B = 16384
_K = 64
_NC = 2
_NS = 16
_NW = _NC * _NS
_BPW = _B // _NW
_CH = 128
_NCH = _BPW // _CH
_L = 16

_PARAMS = pltpu.CompilerParams(needs_layout_passes=False, use_tc_tiling_on_sc=False)


def _worker_base():
    wid = lax.axis_index("s") * _NC + lax.axis_index("c")
    return wid * _BPW


def _load_index_chunks(idx_h, base, idxv):
    for c in range(_NCH):
        pltpu.sync_copy(idx_h.at[pl.ds(base + c * _CH, _CH)], idxv.at[c])


def _bias_body(uid_h, iid_h, ub_h, ib_h, out_h, uidx, iidx, ubg, ibg, outv, sem):
    base = _worker_base()
    _load_index_chunks(uid_h, base, uidx)
    _load_index_chunks(iid_h, base, iidx)
    cps = []
    for c in range(_NCH):
        cps.append(pltpu.async_copy(ub_h.at[uidx.at[c]], ubg.at[pl.ds(c * _CH, _CH)], sem))
        cps.append(pltpu.async_copy(ib_h.at[iidx.at[c]], ibg.at[pl.ds(c * _CH, _CH)], sem))
    for cp in cps:
        cp.wait()

    def body(i, carry):
        outv[pl.ds(i * _L, _L)] = ubg[pl.ds(i * _L, _L)] + ibg[pl.ds(i * _L, _L)]
        return carry

    lax.fori_loop(0, _BPW // _L, body, 0)
    pltpu.sync_copy(outv, out_h.at[pl.ds(base, _BPW)])


def _gather_body(idx_h, tab_h, out_h, idxv, rows, sem):
    base = _worker_base()
    _load_index_chunks(idx_h, base, idxv)
    cps = []
    for c in range(_NCH):
        cps.append(pltpu.async_copy(tab_h.at[idxv.at[c]], rows.at[pl.ds(c * _CH, _CH)], sem))
    for cp in cps:
        cp.wait()
    pltpu.sync_copy(rows, out_h.at[pl.ds(base, _BPW)])


def _combine_body(su_h, si_h, bs_h, out_h, su, si, bsv, outv, sem):
    base = _worker_base()
    cu = pltpu.async_copy(su_h.at[pl.ds(base, _BPW)], su, sem)
    ci = pltpu.async_copy(si_h.at[pl.ds(base, _BPW)], si, sem)
    pltpu.sync_copy(bs_h.at[pl.ds(base, _BPW)], bsv)
    cu.wait()
    ci.wait()

    def group(g, carry):
        r0 = g * _L
        rows16 = r0 + lax.iota(jnp.int32, _L)
        acc = bsv[pl.ds(r0, _L)]
        for j in range(_K):
            cols = jnp.full((_L,), j, jnp.int32)
            acc = acc + (plsc.load_gather(su, [rows16, cols])
                         * plsc.load_gather(si, [rows16, cols]))
        outv[pl.ds(r0, _L)] = acc
        return carry

    lax.fori_loop(0, _BPW // _L, group, 0)
    pltpu.sync_copy(outv, out_h.at[pl.ds(base, _BPW)])


def kernel(user_id, item_id, user_factors, item_factors, user_bias, item_bias):
    uid = user_id.reshape(_B)
    iid = item_id.reshape(_B)
    mesh = plsc.VectorSubcoreMesh(core_axis_name="c", subcore_axis_name="s")

    idx_scr = pltpu.VMEM((_NCH, _CH), jnp.int32)

    bias_call = pl.kernel(
        _bias_body,
        out_type=jax.ShapeDtypeStruct((_B,), jnp.float32),
        mesh=mesh,
        scratch_types=[
            idx_scr, idx_scr,
            pltpu.VMEM((_BPW,), jnp.float32),
            pltpu.VMEM((_BPW,), jnp.float32),
            pltpu.VMEM((_BPW,), jnp.float32),
            pltpu.SemaphoreType.DMA,
        ],
        compiler_params=_PARAMS,
    )
    bias_sum = bias_call(uid, iid, user_bias.reshape(-1), item_bias.reshape(-1))

    gather_call = pl.kernel(
        _gather_body,
        out_type=jax.ShapeDtypeStruct((_B, _K), jnp.float32),
        mesh=mesh,
        scratch_types=[
            idx_scr,
            pltpu.VMEM((_BPW, _K), jnp.float32),
            pltpu.SemaphoreType.DMA,
        ],
        compiler_params=_PARAMS,
    )
    su = gather_call(uid, user_factors)
    si = gather_call(iid, item_factors)

    combine_call = pl.kernel(
        _combine_body,
        out_type=jax.ShapeDtypeStruct((_B,), jnp.float32),
        mesh=mesh,
        scratch_types=[
            pltpu.VMEM((_BPW, _K), jnp.float32),
            pltpu.VMEM((_BPW, _K), jnp.float32),
            pltpu.VMEM((_BPW,), jnp.float32),
            pltpu.VMEM((_BPW,), jnp.float32),
            pltpu.SemaphoreType.DMA,
        ],
        compiler_params=_PARAMS,
    )
    return combine_call(su, si, bias_sum)
